# baseline (speedup 1.0000x reference)
import functools

import jax
import jax.numpy as jnp
from jax import lax
from jax.experimental import pallas as pl
from jax.experimental.pallas import tpu as pltpu
from jax.experimental.pallas import tpu_sc as plsc

N = 16384
F = 8
NUM_TYPES = 8
EMBED_DIM = 256

_INFO = plsc.get_sparse_core_info()
NC, NS, L = _INFO.num_cores, _INFO.num_subcores, _INFO.num_lanes
NW = NC * NS
B_PER_W = N // NW
CHUNK = 256
N_CHUNKS = B_PER_W // CHUNK


def _make_kernel():
  mesh = plsc.VectorSubcoreMesh(core_axis_name="c", subcore_axis_name="s")

  @functools.partial(
      pl.kernel,
      mesh=mesh,
      compiler_params=pltpu.CompilerParams(needs_layout_passes=False),
      out_type=jax.ShapeDtypeStruct((N, EMBED_DIM), jnp.float32),
      scratch_types=[
          pltpu.VMEM((CHUNK * F,), jnp.float32),
          pltpu.VMEM((CHUNK,), jnp.int32),
          pltpu.VMEM((CHUNK, EMBED_DIM), jnp.float32),
          pltpu.SemaphoreType.DMA,
      ],
  )
  def k(inputs_flat_hbm, table_hbm, out_hbm, vals_v, idx_v, rows_v, sem):
    wid = lax.axis_index("s") * NC + lax.axis_index("c")
    base = wid * B_PER_W

    for chunk in range(N_CHUNKS):
      row0 = base + chunk * CHUNK
      pltpu.sync_copy(inputs_flat_hbm.at[pl.ds(row0 * F, CHUNK * F)], vals_v)
      lanes = lax.iota(jnp.int32, L)
      for i in range(CHUNK // L):
        pos = lanes * F + (i * L * F + 2)
        col2 = plsc.load_gather(vals_v, [pos])
        idx = jnp.clip(col2.astype(jnp.int32), 0, NUM_TYPES - 1)
        idx_v[pl.ds(i * L, L)] = idx
      pltpu.async_copy(table_hbm.at[idx_v], rows_v, sem).wait()
      pltpu.sync_copy(rows_v, out_hbm.at[pl.ds(row0, CHUNK)])

  return k


_kernel_call = _make_kernel()


@jax.jit
def kernel(inputs, type_embed):
  if inputs.ndim == 3:
    inputs = inputs[0]
  return _kernel_call(inputs.reshape(-1), type_embed)
